# Initial kernel scaffold; baseline (speedup 1.0000x reference)
import jax
import jax.numpy as jnp
from jax.experimental import pallas as pl


def kernel(x, edge_index, cc_mask, W1L, W1H, W2L, W2H, WX, lam, lin_w, lin_b):
    raise NotImplementedError("write your pallas kernel here")


# speedup vs baseline: 13.9687x; 13.9687x over previous
import functools

import jax
import jax.numpy as jnp
from jax import lax
from jax.experimental import pallas as pl
from jax.experimental.pallas import tpu as pltpu
from jax.experimental.pallas import tpu_sc as plsc

N = 10000
IN = 128
HID = 256
OUT = 128
E = 320000

NC = 2
NS = 16
L = 16
K = 128

CH_PREP = 79
NCH = NC * NS * CH_PREP
EP = NCH * K
CH_SPMM = NCH // NS
BN = 400
NR = 22400
RPT = NR // NS
RQ = 200
DUMMY = 2 * N

_SC_PARAMS = pltpu.CompilerParams(
    needs_layout_passes=False, use_tc_tiling_on_sc=False)

_mesh = functools.partial(
    plsc.VectorSubcoreMesh, core_axis_name="c", subcore_axis_name="s",
    num_cores=NC, num_subcores=NS)


def _prep_body(r_hbm, c_hbm, mask_hbm, d_hbm, deg_hbm,
               mask_v, rv, cv, dv, ones_v, row_v, deg_sh, sem):
    cid = lax.axis_index("c")
    sid = lax.axis_index("s")
    wid = cid * NS + sid

    pltpu.sync_copy(mask_hbm, mask_v)

    zero16 = jnp.zeros((L,), jnp.float32)
    one16 = jnp.ones((L,), jnp.float32)

    def _zrow(i, _):
        row_v[i, :] = zero16
        return 0
    lax.fori_loop(0, RPT, _zrow, 0)
    pltpu.sync_copy(row_v, deg_sh.at[pl.ds(sid * RPT, RPT)])

    def _ones(i, _):
        ones_v[i, :] = one16
        return 0
    lax.fori_loop(0, K, _ones, 0)
    plsc.subcore_barrier()

    def _chunk(t, _):
        g = wid * CH_PREP + t
        pltpu.sync_copy(r_hbm.at[pl.ds(g * K, K)], rv)
        pltpu.sync_copy(c_hbm.at[pl.ds(g * K, K)], cv)
        for j in range(K // L):
            r16 = rv[pl.ds(j * L, L)]
            c16 = cv[pl.ds(j * L, L)]
            m16 = plsc.load_gather(mask_v, [c16])
            d16 = r16 + N * (1 - m16)
            d16 = jnp.where(r16 == c16, DUMMY, d16)
            dv[pl.ds(j * L, L)] = d16
        pltpu.sync_copy(dv, d_hbm.at[pl.ds(g * K, K)])
        pltpu.sync_copy(ones_v, deg_sh.at[dv], add=True)
        return 0
    lax.fori_loop(0, CH_PREP, _chunk, 0)
    plsc.subcore_barrier()

    pltpu.sync_copy(deg_sh.at[pl.ds(sid * RPT, RPT)], row_v)
    pltpu.sync_copy(row_v, deg_hbm.at[pl.ds(cid * NR + sid * RPT, RPT)])


def _prep(rp, cp, mask):
    return pl.kernel(
        _prep_body,
        out_type=(
            jax.ShapeDtypeStruct((EP,), jnp.int32),
            jax.ShapeDtypeStruct((NC * NR, L), jnp.float32),
        ),
        mesh=_mesh(),
        scratch_types=[
            pltpu.VMEM((N,), jnp.int32),
            pltpu.VMEM((K,), jnp.int32),
            pltpu.VMEM((K,), jnp.int32),
            pltpu.VMEM((K,), jnp.int32),
            pltpu.VMEM((K, L), jnp.float32),
            pltpu.VMEM((RPT, L), jnp.float32),
            pltpu.VMEM_SHARED((NR, L), jnp.float32),
            pltpu.SemaphoreType.DMA,
        ],
        compiler_params=_SC_PARAMS,
    )(rp, cp, mask)


def _spmm_body(tab_hbm, c_hbm, d_hbm, out_hbm, cv, dv, rows_v, buf_v, acc_sh,
               sem):
    cid = lax.axis_index("c")
    sid = lax.axis_index("s")

    zero16 = jnp.zeros((L,), jnp.float32)

    def _zrow(i, _):
        for j in range(64 // L):
            buf_v[i, pl.ds(j * L, L)] = zero16
        return 0
    lax.fori_loop(0, RQ, _zrow, 0)
    for q in range(RPT // RQ):
        pltpu.sync_copy(buf_v, acc_sh.at[pl.ds(sid * RPT + q * RQ, RQ)])
    plsc.subcore_barrier()

    coff = cid * N

    def _chunk(t, _):
        g = sid * CH_SPMM + t
        pltpu.sync_copy(c_hbm.at[pl.ds(g * K, K)], cv)
        pltpu.sync_copy(d_hbm.at[pl.ds(g * K, K)], dv)
        for j in range(K // L):
            cv[pl.ds(j * L, L)] = cv[pl.ds(j * L, L)] + coff
        pltpu.async_copy(tab_hbm.at[cv], rows_v, sem).wait()
        pltpu.sync_copy(rows_v, acc_sh.at[dv], add=True)
        return 0
    lax.fori_loop(0, CH_SPMM, _chunk, 0)
    plsc.subcore_barrier()

    for q in range(RPT // RQ):
        pltpu.sync_copy(acc_sh.at[pl.ds(sid * RPT + q * RQ, RQ)], buf_v)
        pltpu.sync_copy(buf_v, out_hbm.at[pl.ds(cid * NR + sid * RPT + q * RQ, RQ)])


def _spmm(tab, cp, d_idx):
    return pl.kernel(
        _spmm_body,
        out_type=jax.ShapeDtypeStruct((NC * NR, 64), jnp.float32),
        mesh=_mesh(),
        scratch_types=[
            pltpu.VMEM((K,), jnp.int32),
            pltpu.VMEM((K,), jnp.int32),
            pltpu.VMEM((K, 64), jnp.float32),
            pltpu.VMEM((RQ, 64), jnp.float32),
            pltpu.VMEM_SHARED((NR, 64), jnp.float32),
            pltpu.SemaphoreType.DMA,
        ],
        compiler_params=_SC_PARAMS,
    )(tab, cp, d_idx)


_GB = N // BN
_OFF_N = N // BN
_OFF_SC1 = NR // BN


def _tca_body(dl0, dl1, dh0, dh1, x, mk, za, zb, dl, dh):
    degl = 1.0 + dl0[...] + dl1[...]
    degh = 1.0 + dh0[...] + dh1[...]
    dli = lax.rsqrt(degl[:, :1])
    dhi = lax.rsqrt(degh[:, :1])
    dl[...] = dli
    dh[...] = dhi
    sel = mk[...]
    dsel = sel * dli + (1.0 - sel) * dhi
    z = dsel * x[...]
    za[...] = z[:, :64]
    zb[...] = z[:, 64:]


def _tc_a(deg, x, maskf):
    grid = (_GB,)
    bspec = lambda off: pl.BlockSpec((BN, L), lambda i, o=off: (i + o, 0))
    out = pl.pallas_call(
        _tca_body,
        grid=grid,
        in_specs=[
            bspec(0), bspec(_OFF_SC1), bspec(_OFF_N), bspec(_OFF_N + _OFF_SC1),
            pl.BlockSpec((BN, IN), lambda i: (i, 0)),
            pl.BlockSpec((BN, 1), lambda i: (i, 0)),
        ],
        out_specs=[
            pl.BlockSpec((BN, 64), lambda i: (i, 0)),
            pl.BlockSpec((BN, 64), lambda i: (i, 0)),
            pl.BlockSpec((BN, 1), lambda i: (i, 0)),
            pl.BlockSpec((BN, 1), lambda i: (i, 0)),
        ],
        out_shape=[
            jax.ShapeDtypeStruct((N, 64), jnp.float32),
            jax.ShapeDtypeStruct((N, 64), jnp.float32),
            jax.ShapeDtypeStruct((N, 1), jnp.float32),
            jax.ShapeDtypeStruct((N, 1), jnp.float32),
        ],
    )(deg, deg, deg, deg, x, maskf)
    return out


def _tcb_body(al0, al1, ah0, ah1, x, mk, dl, dh, w1l, w1h,
              hl, hh, z2a, z2b, z2c, z2d):
    dli = dl[...]
    dhi = dh[...]
    xx = x[...]
    sl = dli * (jnp.concatenate([al0[...], al1[...]], axis=1) + dli * xx)
    sh = dhi * (jnp.concatenate([ah0[...], ah1[...]], axis=1) + dhi * xx)
    hlv = jnp.maximum(jnp.dot(sl, w1l[...], preferred_element_type=jnp.float32), 0.0)
    hhv = jnp.maximum(jnp.dot(sh, w1h[...], preferred_element_type=jnp.float32), 0.0)
    hl[...] = hlv
    hh[...] = hhv
    sel = mk[...]
    dsel = sel * dli + (1.0 - sel) * dhi
    z2 = dsel * (sel * hlv + (1.0 - sel) * hhv)
    z2a[...] = z2[:, 0:64]
    z2b[...] = z2[:, 64:128]
    z2c[...] = z2[:, 128:192]
    z2d[...] = z2[:, 192:256]


def _tc_b(acc1, x, maskf, dl, dh, w1l, w1h):
    grid = (_GB,)
    aspec = lambda off: pl.BlockSpec((BN, 64), lambda i, o=off: (i + o, 0))
    z64 = jax.ShapeDtypeStruct((N, 64), jnp.float32)
    return pl.pallas_call(
        _tcb_body,
        grid=grid,
        in_specs=[
            aspec(0), aspec(_OFF_SC1), aspec(_OFF_N), aspec(_OFF_N + _OFF_SC1),
            pl.BlockSpec((BN, IN), lambda i: (i, 0)),
            pl.BlockSpec((BN, 1), lambda i: (i, 0)),
            pl.BlockSpec((BN, 1), lambda i: (i, 0)),
            pl.BlockSpec((BN, 1), lambda i: (i, 0)),
            pl.BlockSpec((IN, HID), lambda i: (0, 0)),
            pl.BlockSpec((IN, HID), lambda i: (0, 0)),
        ],
        out_specs=[
            pl.BlockSpec((BN, HID), lambda i: (i, 0)),
            pl.BlockSpec((BN, HID), lambda i: (i, 0)),
            pl.BlockSpec((BN, 64), lambda i: (i, 0)),
            pl.BlockSpec((BN, 64), lambda i: (i, 0)),
            pl.BlockSpec((BN, 64), lambda i: (i, 0)),
            pl.BlockSpec((BN, 64), lambda i: (i, 0)),
        ],
        out_shape=[
            jax.ShapeDtypeStruct((N, HID), jnp.float32),
            jax.ShapeDtypeStruct((N, HID), jnp.float32),
            z64, z64, z64, z64,
        ],
    )(acc1, acc1, acc1, acc1, x, maskf, dl, dh, w1l, w1h)


def _tcc_body(al0, al1, al2, al3, ah0, ah1, ah2, ah3, hl, hh, dl, dh,
              w2l, w2h, lwt, lb, out):
    dli = dl[...]
    dhi = dh[...]
    s2l = dli * (jnp.concatenate(
        [al0[...], al1[...], al2[...], al3[...]], axis=1) + dli * hl[...])
    s2h = dhi * (jnp.concatenate(
        [ah0[...], ah1[...], ah2[...], ah3[...]], axis=1) + dhi * hh[...])
    u = jnp.dot(s2l, w2l[...], preferred_element_type=jnp.float32)
    u = u + jnp.dot(s2h, w2h[...], preferred_element_type=jnp.float32)
    u = jnp.maximum(u, 0.0)
    out[...] = jnp.dot(u, lwt[...], preferred_element_type=jnp.float32) + lb[...]


def _tc_c(acc2a, acc2b, hl, hh, dl, dh, w2ls, w2hs, lin_wt, lin_b2):
    grid = (_GB,)
    aspec = lambda off: pl.BlockSpec((BN, 64), lambda i, o=off: (i + o, 0))
    return pl.pallas_call(
        _tcc_body,
        grid=grid,
        in_specs=[
            aspec(0), aspec(_OFF_SC1), aspec(0), aspec(_OFF_SC1),
            aspec(_OFF_N), aspec(_OFF_N + _OFF_SC1),
            aspec(_OFF_N), aspec(_OFF_N + _OFF_SC1),
            pl.BlockSpec((BN, HID), lambda i: (i, 0)),
            pl.BlockSpec((BN, HID), lambda i: (i, 0)),
            pl.BlockSpec((BN, 1), lambda i: (i, 0)),
            pl.BlockSpec((BN, 1), lambda i: (i, 0)),
            pl.BlockSpec((HID, HID), lambda i: (0, 0)),
            pl.BlockSpec((HID, HID), lambda i: (0, 0)),
            pl.BlockSpec((HID, OUT), lambda i: (0, 0)),
            pl.BlockSpec((1, OUT), lambda i: (0, 0)),
        ],
        out_specs=pl.BlockSpec((BN, OUT), lambda i: (i, 0)),
        out_shape=jax.ShapeDtypeStruct((N, OUT), jnp.float32),
    )(acc2a, acc2a, acc2b, acc2b, acc2a, acc2a, acc2b, acc2b,
      hl, hh, dl, dh, w2ls, w2hs, lin_wt, lin_b2)


def kernel(x, edge_index, cc_mask, W1L, W1H, W2L, W2H, WX, lam, lin_w, lin_b):
    del WX
    row = edge_index[1]
    col = edge_index[0]
    pad = EP - E
    rp = jnp.concatenate([row, jnp.zeros((pad,), jnp.int32)])
    cp = jnp.concatenate([col, jnp.zeros((pad,), jnp.int32)])
    maskf = cc_mask.astype(jnp.float32)[:, None]

    d_idx, deg = _prep(rp, cp, cc_mask)

    za, zb, dl, dh = _tc_a(deg, x, maskf)
    z1 = jnp.concatenate([za, zb], axis=0)

    acc1 = _spmm(z1, cp, d_idx)

    hl, hh, z2a, z2b, z2c, z2d = _tc_b(acc1, x, maskf, dl, dh, W1L, W1H)

    acc2a = _spmm(jnp.concatenate([z2a, z2b], axis=0), cp, d_idx)
    acc2b = _spmm(jnp.concatenate([z2c, z2d], axis=0), cp, d_idx)

    lw = jax.nn.softmax(lam)
    w2ls = lw[1] * W2L
    w2hs = (lw[0] + lw[2]) * W2H
    lin_wt = lin_w.T
    lin_b2 = lin_b[None, :]

    return _tc_c(acc2a, acc2b, hl, hh, dl, dh, w2ls, w2hs, lin_wt, lin_b2)
